# baseline (speedup 1.0000x reference)
import jax
import jax.numpy as jnp
from jax.experimental import pallas as pl
from jax.experimental.pallas import tpu as pltpu

E = 64
H = 1024
MOE_I = 512
SHARED_I = 4096
T = 64
SBLK = 512
NSH = SHARED_I // SBLK
MH = MOE_I // 2
SH2 = SBLK // 2


def _dot_t(a, b):
    return jax.lax.dot_general(a, b, (((1,), (1,)), ((), ())),
                               preferred_element_type=jnp.float32)


def _router_body(x_ref, gw_ref, idx_ref):
    logits = _dot_t(x_ref[...], gw_ref[...])
    m = jnp.max(logits, axis=1, keepdims=True)
    eiota = jax.lax.broadcasted_iota(jnp.int32, (T, E), 1)
    cand = jnp.where(logits >= m, eiota, E)
    idx_ref[...] = jnp.min(cand, axis=1, keepdims=True)


def _fused_body(order_ref, n_ref, x_ref, top1_ref,
                sg0_ref, sg1_ref, su0_ref, su1_ref, sd0_ref, sd1_ref,
                wg0_ref, wg1_ref, wu0_ref, wu1_ref, wd0_ref, wd1_ref,
                out_ref):
    i = pl.program_id(0)

    @pl.when(i == 0)
    def _():
        out_ref[...] = jnp.zeros_like(out_ref)

    @pl.when(i < NSH)
    def _():
        x = x_ref[...]
        a0 = jax.nn.silu(_dot_t(x, sg0_ref[...])) * _dot_t(x, su0_ref[...])
        a1 = jax.nn.silu(_dot_t(x, sg1_ref[...])) * _dot_t(x, su1_ref[...])
        out_ref[...] += _dot_t(a0, sd0_ref[...]) + _dot_t(a1, sd1_ref[...])

    @pl.when((i >= NSH) & (i - NSH < n_ref[0]))
    def _():
        e = order_ref[i - NSH]
        x = x_ref[...]
        a0 = jax.nn.silu(_dot_t(x, wg0_ref[0])) * _dot_t(x, wu0_ref[0])
        a1 = jax.nn.silu(_dot_t(x, wg1_ref[0])) * _dot_t(x, wu1_ref[0])
        o = _dot_t(a0, wd0_ref[0]) + _dot_t(a1, wd1_ref[0])
        mask = (top1_ref[...] == e).astype(jnp.float32)
        out_ref[...] += o * mask


def kernel(hidden_states, gate_w, expert_gate_w, expert_up_w, expert_down_w,
           shared_gate_w, shared_up_w, shared_down_w):
    bsz, seq_len, hidden = hidden_states.shape
    x = hidden_states.reshape(T, H)

    top1 = pl.pallas_call(
        _router_body,
        out_shape=jax.ShapeDtypeStruct((T, 1), jnp.int32),
    )(x, gate_w)

    idx = top1[:, 0]
    active = jnp.zeros((E,), jnp.int32).at[idx].set(1)
    n = jnp.sum(active).astype(jnp.int32)
    order = jnp.argsort(1 - active).astype(jnp.int32)
    last = order[jnp.maximum(n - 1, 0)]
    order = jnp.where(jnp.arange(E, dtype=jnp.int32) < n, order, last)

    def _shj(i):
        return jnp.minimum(i, NSH - 1)

    def _exj(i, order):
        return order[jnp.maximum(i - NSH, 0)]

    out = pl.pallas_call(
        _fused_body,
        grid_spec=pltpu.PrefetchScalarGridSpec(
            num_scalar_prefetch=2,
            grid=(NSH + E,),
            in_specs=[
                pl.BlockSpec((T, H), lambda i, o, nn: (0, 0)),
                pl.BlockSpec((T, 1), lambda i, o, nn: (0, 0)),
                pl.BlockSpec((SH2, H), lambda i, o, nn: (2 * _shj(i), 0)),
                pl.BlockSpec((SH2, H), lambda i, o, nn: (2 * _shj(i) + 1, 0)),
                pl.BlockSpec((SH2, H), lambda i, o, nn: (2 * _shj(i), 0)),
                pl.BlockSpec((SH2, H), lambda i, o, nn: (2 * _shj(i) + 1, 0)),
                pl.BlockSpec((H, SH2), lambda i, o, nn: (0, 2 * _shj(i))),
                pl.BlockSpec((H, SH2), lambda i, o, nn: (0, 2 * _shj(i) + 1)),
                pl.BlockSpec((1, MH, H), lambda i, o, nn: (_exj(i, o), 0, 0)),
                pl.BlockSpec((1, MH, H), lambda i, o, nn: (_exj(i, o), 1, 0)),
                pl.BlockSpec((1, MH, H), lambda i, o, nn: (_exj(i, o), 0, 0)),
                pl.BlockSpec((1, MH, H), lambda i, o, nn: (_exj(i, o), 1, 0)),
                pl.BlockSpec((1, H, MH), lambda i, o, nn: (_exj(i, o), 0, 0)),
                pl.BlockSpec((1, H, MH), lambda i, o, nn: (_exj(i, o), 0, 1)),
            ],
            out_specs=pl.BlockSpec((T, H), lambda i, o, nn: (0, 0)),
        ),
        out_shape=jax.ShapeDtypeStruct((T, H), jnp.float32),
    )(order, n.reshape(1), x, top1,
      shared_gate_w, shared_gate_w, shared_up_w, shared_up_w,
      shared_down_w, shared_down_w,
      expert_gate_w, expert_gate_w, expert_up_w, expert_up_w,
      expert_down_w, expert_down_w)

    return out.reshape(bsz, seq_len, hidden)


# speedup vs baseline: 1.0883x; 1.0883x over previous
import jax
import jax.numpy as jnp
from jax.experimental import pallas as pl
from jax.experimental.pallas import tpu as pltpu

E = 64
H = 1024
MOE_I = 512
SHARED_I = 4096
T = 64
SBLK = 512
NSH = SHARED_I // SBLK


def _dot_t(a, b):
    return jax.lax.dot_general(a, b, (((1,), (1,)), ((), ())),
                               preferred_element_type=jnp.float32)


def _shared_body(x_ref, gw_ref, sg_ref, su_ref, sd_ref,
                 out_ref, top1_ref, order_ref, n_ref):
    j = pl.program_id(0)

    @pl.when(j == 0)
    def _():
        out_ref[...] = jnp.zeros_like(out_ref)
        x = x_ref[...]
        gw = gw_ref[...]
        lg = _dot_t(x, gw)
        lgt = jax.lax.dot_general(gw, x, (((1,), (1,)), ((), ())),
                                  preferred_element_type=jnp.float32)
        m1 = jnp.max(lg, axis=1, keepdims=True)
        cand1 = jnp.where(lg >= m1, jax.lax.broadcasted_iota(jnp.int32, (T, E), 1), E)
        top1_ref[...] = jnp.min(cand1, axis=1, keepdims=True)
        m0 = jnp.max(lgt, axis=0, keepdims=True)
        cand0 = jnp.where(lgt >= m0, jax.lax.broadcasted_iota(jnp.int32, (E, T), 0), E)
        idx_row = jnp.min(cand0, axis=0, keepdims=True)
        ohT = (jax.lax.broadcasted_iota(jnp.int32, (E, T), 0) == idx_row
               ).astype(jnp.float32)
        active = jnp.max(ohT, axis=1, keepdims=True)
        etri = (jax.lax.broadcasted_iota(jnp.int32, (E, E), 1)
                <= jax.lax.broadcasted_iota(jnp.int32, (E, E), 0)
                ).astype(jnp.float32)
        pos = jax.lax.dot_general(etri, active, (((1,), (0,)), ((), ())),
                                  preferred_element_type=jnp.float32)
        nact = jnp.sum(active, axis=0, keepdims=True)
        slot = pos - 1.0
        jio = jax.lax.broadcasted_iota(jnp.int32, (E, E), 1).astype(jnp.float32)
        order_oh = active * (slot == jio).astype(jnp.float32)
        evals = jax.lax.broadcasted_iota(jnp.int32, (E, 1), 0).astype(jnp.float32)
        order_row = jax.lax.dot_general(
            order_oh, evals, (((0,), (0,)), ((), ())),
            preferred_element_type=jnp.float32)
        lastid = jnp.max(evals * active, axis=0, keepdims=True)
        sio = jax.lax.broadcasted_iota(jnp.int32, (E, 1), 0).astype(jnp.float32)
        padded = jnp.where(sio < nact, order_row, lastid)
        order_ref[...] = padded.astype(jnp.int32)
        n_ref[...] = nact.astype(jnp.int32)

    x = x_ref[...]
    g = _dot_t(x, sg_ref[...])
    u = _dot_t(x, su_ref[...])
    act = jax.nn.silu(g) * u
    out_ref[...] += _dot_t(act, sd_ref[...])


def _moe_body(order_ref, n_ref, x_ref, top1_ref, shared_ref,
              wg_ref, wu_ref, wd_ref, out_ref):
    i = pl.program_id(0)

    @pl.when(i == 0)
    def _():
        out_ref[...] = shared_ref[...]

    @pl.when(i < n_ref[0])
    def _():
        e = order_ref[i]
        x = x_ref[...]
        g = _dot_t(x, wg_ref[0])
        u = _dot_t(x, wu_ref[0])
        act = jax.nn.silu(g) * u
        o = _dot_t(act, wd_ref[0])
        mask = (top1_ref[...] == e).astype(jnp.float32)
        out_ref[...] += o * mask


def kernel(hidden_states, gate_w, expert_gate_w, expert_up_w, expert_down_w,
           shared_gate_w, shared_up_w, shared_down_w):
    bsz, seq_len, hidden = hidden_states.shape
    x = hidden_states.reshape(T, H)

    shared_out, top1, order2d, n2d = pl.pallas_call(
        _shared_body,
        grid=(NSH,),
        in_specs=[
            pl.BlockSpec((T, H), lambda j: (0, 0)),
            pl.BlockSpec((E, H), lambda j: (0, 0)),
            pl.BlockSpec((SBLK, H), lambda j: (j, 0)),
            pl.BlockSpec((SBLK, H), lambda j: (j, 0)),
            pl.BlockSpec((H, SBLK), lambda j: (0, j)),
        ],
        out_specs=[
            pl.BlockSpec((T, H), lambda j: (0, 0)),
            pl.BlockSpec((T, 1), lambda j: (0, 0)),
            pl.BlockSpec((E, 1), lambda j: (0, 0)),
            pl.BlockSpec((1, 1), lambda j: (0, 0)),
        ],
        out_shape=[
            jax.ShapeDtypeStruct((T, H), jnp.float32),
            jax.ShapeDtypeStruct((T, 1), jnp.int32),
            jax.ShapeDtypeStruct((E, 1), jnp.int32),
            jax.ShapeDtypeStruct((1, 1), jnp.int32),
        ],
    )(x, gate_w, shared_gate_w, shared_up_w, shared_down_w)

    order = order2d.reshape(E)
    n = n2d.reshape(1)

    out = pl.pallas_call(
        _moe_body,
        grid_spec=pltpu.PrefetchScalarGridSpec(
            num_scalar_prefetch=2,
            grid=(E,),
            in_specs=[
                pl.BlockSpec((T, H), lambda i, order, nn: (0, 0)),
                pl.BlockSpec((T, 1), lambda i, order, nn: (0, 0)),
                pl.BlockSpec((T, H), lambda i, order, nn: (0, 0)),
                pl.BlockSpec((1, MOE_I, H), lambda i, order, nn: (order[i], 0, 0)),
                pl.BlockSpec((1, MOE_I, H), lambda i, order, nn: (order[i], 0, 0)),
                pl.BlockSpec((1, H, MOE_I), lambda i, order, nn: (order[i], 0, 0)),
            ],
            out_specs=pl.BlockSpec((T, H), lambda i, order, nn: (0, 0)),
        ),
        out_shape=jax.ShapeDtypeStruct((T, H), jnp.float32),
    )(order, n, x, top1, shared_out,
      expert_gate_w, expert_up_w, expert_down_w)

    return out.reshape(bsz, seq_len, hidden)
